# baseline (speedup 1.0000x reference)
import functools

import jax
import jax.numpy as jnp
from jax import lax
from jax.experimental import pallas as pl
from jax.experimental.pallas import tpu as pltpu
from jax.experimental.pallas import tpu_sc as plsc

B, S, D = 1, 2048, 1024
H, KVH, HD = 16, 8, 64
E, FF = 8, 2048
EPS = 1e-05
JITTER = 0.01
NEG = -1e30

TBLK = 256
NT = S // TBLK
FBLK = 512
NF = FF // FBLK


def _rmsnorm(x, w):
    return x * jax.lax.rsqrt(jnp.mean(x * x, axis=-1, keepdims=True) + EPS) * w


def _rope(x, cos, sin):
    rot = jnp.concatenate([-x[:, HD // 2:], x[:, :HD // 2]], axis=1)
    return x * cos + rot * sin


def _k1_body(x_ref, w_ref, wqkv_ref, out_ref):
    h = _rmsnorm(x_ref[...], w_ref[...])
    out_ref[...] = jnp.dot(h, wqkv_ref[...], preferred_element_type=jnp.float32)


def _qkv(x, ln1_w, wqkv):
    return pl.pallas_call(
        _k1_body,
        grid=(NT,),
        in_specs=[
            pl.BlockSpec((TBLK, D), lambda i: (i, 0)),
            pl.BlockSpec((1, D), lambda i: (0, 0)),
            pl.BlockSpec((D, (H + 2 * KVH) * HD), lambda i: (0, 0)),
        ],
        out_specs=pl.BlockSpec((TBLK, (H + 2 * KVH) * HD), lambda i: (i, 0)),
        out_shape=jax.ShapeDtypeStruct((S, (H + 2 * KVH) * HD), jnp.float32),
    )(x, ln1_w.reshape(1, D), wqkv)


KBLK = 512
NKB = S // KBLK
_JR = KBLK // TBLK


def _k2_body(q_ref, k_ref, v_ref, cosq_ref, sinq_ref, cosk_ref, sink_ref,
             o_ref, acc_ref, m_ref, l_ref):
    i = pl.program_id(1)
    j = pl.program_id(2)
    jlast = i // _JR

    @pl.when(j == 0)
    def _():
        m_ref[...] = jnp.full_like(m_ref, NEG)
        l_ref[...] = jnp.zeros_like(l_ref)

    @pl.when(j <= jlast)
    def _():
        q = _rope(q_ref[0], cosq_ref[...], sinq_ref[...]) * (1.0 / (HD ** 0.5))
        k = _rope(k_ref[0], cosk_ref[...], sink_ref[...])
        logits = jax.lax.dot_general(
            q, k, (((1,), (1,)), ((), ())), preferred_element_type=jnp.float32)
        qi = i * TBLK + jax.lax.broadcasted_iota(jnp.int32, (TBLK, KBLK), 0)
        kj = j * KBLK + jax.lax.broadcasted_iota(jnp.int32, (TBLK, KBLK), 1)
        logits = jnp.where(kj <= qi, logits, NEG)
        m_prev = m_ref[:, 0:1]
        m_new = jnp.maximum(m_prev, jnp.max(logits, axis=-1, keepdims=True))
        p = jnp.exp(logits - m_new)
        alpha = jnp.exp(m_prev - m_new)
        l_new = alpha * l_ref[:, 0:1] + jnp.sum(p, axis=-1, keepdims=True)
        pv = jnp.dot(p, v_ref[0], preferred_element_type=jnp.float32)

        @pl.when(j == 0)
        def _():
            acc_ref[...] = pv

        @pl.when(j > 0)
        def _():
            acc_ref[...] = acc_ref[...] * alpha + pv

        m_ref[...] = jnp.broadcast_to(m_new, m_ref.shape)
        l_ref[...] = jnp.broadcast_to(l_new, l_ref.shape)

        @pl.when(j == jlast)
        def _():
            o_ref[0] = acc_ref[...] / l_ref[:, 0:1]


def _attention(q3, k3, v3, cos, sin):
    kvmap = lambda h, i, j: (h // (H // KVH), jnp.minimum(j, i // _JR), 0)
    ckmap = lambda h, i, j: (jnp.minimum(j, i // _JR), 0)
    return pl.pallas_call(
        _k2_body,
        grid=(H, NT, NKB),
        in_specs=[
            pl.BlockSpec((1, TBLK, HD), lambda h, i, j: (h, i, 0)),
            pl.BlockSpec((1, KBLK, HD), kvmap),
            pl.BlockSpec((1, KBLK, HD), kvmap),
            pl.BlockSpec((TBLK, HD), lambda h, i, j: (i, 0)),
            pl.BlockSpec((TBLK, HD), lambda h, i, j: (i, 0)),
            pl.BlockSpec((KBLK, HD), ckmap),
            pl.BlockSpec((KBLK, HD), ckmap),
        ],
        out_specs=pl.BlockSpec((1, TBLK, HD), lambda h, i, j: (h, i, 0)),
        out_shape=jax.ShapeDtypeStruct((H, S, HD), jnp.float32),
        scratch_shapes=[
            pltpu.VMEM((TBLK, HD), jnp.float32),
            pltpu.VMEM((TBLK, 128), jnp.float32),
            pltpu.VMEM((TBLK, 128), jnp.float32),
        ],
    )(q3, k3, v3, cos, sin, cos, sin)


def _k3_body(o_ref, wo_ref, res_ref, w2_ref, gw_ref, r2_ref, xm_ref, lg_ref):
    r2 = jnp.dot(o_ref[...], wo_ref[...], preferred_element_type=jnp.float32) + res_ref[...]
    r2_ref[...] = r2
    h2 = _rmsnorm(r2, w2_ref[...])
    xm_ref[...] = h2
    lg_ref[...] = jnp.dot(h2, gw_ref[...], preferred_element_type=jnp.float32)


def _oproj_router(o2d, wo, resid, ln2_w, gate_w):
    return pl.pallas_call(
        _k3_body,
        grid=(NT,),
        in_specs=[
            pl.BlockSpec((TBLK, H * HD), lambda i: (i, 0)),
            pl.BlockSpec((H * HD, D), lambda i: (0, 0)),
            pl.BlockSpec((TBLK, D), lambda i: (i, 0)),
            pl.BlockSpec((1, D), lambda i: (0, 0)),
            pl.BlockSpec((D, E), lambda i: (0, 0)),
        ],
        out_specs=[
            pl.BlockSpec((TBLK, D), lambda i: (i, 0)),
            pl.BlockSpec((TBLK, D), lambda i: (i, 0)),
            pl.BlockSpec((TBLK, E), lambda i: (i, 0)),
        ],
        out_shape=[
            jax.ShapeDtypeStruct((S, D), jnp.float32),
            jax.ShapeDtypeStruct((S, D), jnp.float32),
            jax.ShapeDtypeStruct((S, E), jnp.float32),
        ],
    )(o2d, wo, resid, ln2_w.reshape(1, D), gate_w)


NA = 2 * S
MBLK = 128
NB = 40
PADT = NB * MBLK
CHUNK = 128
NCH = NA // CHUNK


def _sparsemixer_t(sc):
    iota_e = lax.broadcasted_iota(jnp.int32, (E, CHUNK), 0)
    mlt = jnp.max(sc, axis=0, keepdims=True)
    idx1 = jnp.min(jnp.where(sc == mlt, iota_e, E), axis=0, keepdims=True)
    oh1 = iota_e == idx1
    factor = jnp.maximum(jnp.abs(sc), mlt)
    mask = ((mlt - sc) / factor) > (2.0 * JITTER)
    mg = jnp.where(mask, NEG, sc)
    p1 = jnp.exp(mg - jnp.max(mg, axis=0, keepdims=True))
    sm1 = p1 / jnp.sum(p1, axis=0, keepdims=True)
    mult1 = jnp.sum(jnp.where(oh1, sm1, 0.0), axis=0, keepdims=True)

    msc = jnp.where(oh1, NEG, sc)
    mlt2 = jnp.max(msc, axis=0, keepdims=True)
    idx2 = jnp.min(jnp.where(msc == mlt2, iota_e, E), axis=0, keepdims=True)
    oh2 = iota_e == idx2
    factor2 = jnp.maximum(jnp.abs(sc), mlt2)
    mask2 = ((mlt2 - sc) / factor2) > (2.0 * JITTER)
    mg2 = jnp.where(mask2, NEG, msc)
    p2 = jnp.exp(mg2 - jnp.max(mg2, axis=0, keepdims=True))
    sm2 = p2 / jnp.sum(p2, axis=0, keepdims=True)
    mult2 = jnp.sum(jnp.where(oh2, sm2, 0.0), axis=0, keepdims=True)
    return oh1.astype(jnp.float32), oh2.astype(jnp.float32), mult1, mult2


def _k4b_body(sct_ref, pos_ref, mult_ref, be_ref, cnt_ref, pst_ref):
    p = pl.program_id(0)
    c = pl.program_id(1)
    oh1, oh2, mult1, mult2 = _sparsemixer_t(sct_ref[...])
    is1 = c < (NCH // 2)
    mc = jnp.where(is1, oh1, oh2)
    multc = jnp.where(is1, mult1, mult2)
    colsum = jnp.sum(mc, axis=1, keepdims=True)

    @pl.when(c == 0)
    def _():
        cnt_ref[...] = jnp.zeros_like(cnt_ref)

    @pl.when(p == 0)
    def _():
        cnt_ref[...] += jnp.broadcast_to(colsum, (E, CHUNK))

        @pl.when(c == NCH - 1)
        def _():
            blk = jnp.floor((cnt_ref[...] + (MBLK - 1.0)) * (1.0 / MBLK))
            acc = jnp.zeros((1, CHUNK), jnp.float32)
            for e in range(E):
                pst_ref[e:e + 1, :] = acc
                acc = acc + blk[e:e + 1, :]
            iota_b = lax.broadcasted_iota(jnp.int32, (1, CHUNK), 1)
            bev = jnp.zeros((1, CHUNK), jnp.int32)
            for e in range(E):
                incl = pst_ref[e:e + 1, 0:1] + blk[e:e + 1, 0:1]
                incl_b = jnp.broadcast_to(incl, (1, CHUNK)).astype(jnp.int32)
                bev = bev + jnp.where(iota_b >= incl_b, 1, 0)
            be_ref[...] = jnp.minimum(bev, E - 1)

    @pl.when(p == 1)
    def _():
        iota_r = lax.broadcasted_iota(jnp.int32, (CHUNK, CHUNK), 0)
        iota_c = lax.broadcasted_iota(jnp.int32, (CHUNK, CHUNK), 1)
        tri = jnp.where(iota_r < iota_c, 1.0, 0.0)
        prefix = jnp.dot(mc, tri, preferred_element_type=jnp.float32)
        rank = prefix + jnp.broadcast_to(cnt_ref[:, 0:1], (E, CHUNK))
        base = pst_ref[...] * float(MBLK)
        posv = jnp.sum(mc * (rank + base), axis=0, keepdims=True)
        pos_ref[0] = posv.astype(jnp.int32)
        mult_ref[0] = multc
        cnt_ref[...] += jnp.broadcast_to(colsum, (E, CHUNK))


def _route_sort(logits):
    return pl.pallas_call(
        _k4b_body,
        grid=(2, NCH),
        in_specs=[pl.BlockSpec((E, CHUNK), lambda p, c: (0, lax.rem(c, NCH // 2)))],
        out_specs=[
            pl.BlockSpec((1, 1, CHUNK), lambda p, c: (c, 0, 0)),
            pl.BlockSpec((1, 1, CHUNK), lambda p, c: (c, 0, 0)),
            pl.BlockSpec((1, CHUNK), lambda p, c: (0, 0)),
        ],
        out_shape=[
            jax.ShapeDtypeStruct((NCH, 1, CHUNK), jnp.int32),
            jax.ShapeDtypeStruct((NCH, 1, CHUNK), jnp.float32),
            jax.ShapeDtypeStruct((1, CHUNK), jnp.int32),
        ],
        scratch_shapes=[
            pltpu.VMEM((E, CHUNK), jnp.float32),
            pltpu.VMEM((E, CHUNK), jnp.float32),
        ],
    )(logits.T)


NW = 32
DROWS = 64
CROWS = 32


def _sc_dispatch(xm, posf):
    mesh = plsc.VectorSubcoreMesh(core_axis_name="c", subcore_axis_name="s")

    @functools.partial(
        pl.kernel, mesh=mesh,
        out_type=jax.ShapeDtypeStruct((PADT, D), jnp.float32),
        scratch_types=[
            pltpu.VMEM((DROWS, D), jnp.float32),
            pltpu.VMEM((DROWS,), jnp.int32),
            pltpu.SemaphoreType.DMA,
        ],
    )
    def k(xm_hbm, posf_hbm, xs_hbm, rows_v, idx_v, sem):
        wid = lax.axis_index("s") * 2 + lax.axis_index("c")
        for j in range(NA // (NW * DROWS)):
            q = wid * 2 + j
            base_t = lax.rem(q * DROWS, S)
            pltpu.sync_copy(xm_hbm.at[pl.ds(base_t, DROWS)], rows_v)
            pltpu.sync_copy(posf_hbm.at[pl.ds(q * DROWS, DROWS)], idx_v)
            pltpu.async_copy(rows_v, xs_hbm.at[idx_v], sem).wait()

    return k(xm, posf)


def _sc_combine(ys, pos_a, pos_b, m_a_in, m_b_in):
    mesh = plsc.VectorSubcoreMesh(core_axis_name="c", subcore_axis_name="s")

    @functools.partial(
        pl.kernel, mesh=mesh,
        out_type=jax.ShapeDtypeStruct((S, D), jnp.float32),
        scratch_types=[
            pltpu.VMEM((CROWS, D), jnp.float32),
            pltpu.VMEM((CROWS, D), jnp.float32),
            pltpu.VMEM((CROWS,), jnp.int32),
            pltpu.VMEM((CROWS,), jnp.int32),
            pltpu.VMEM((CROWS, 16), jnp.float32),
            pltpu.VMEM((CROWS, 16), jnp.float32),
            pltpu.SemaphoreType.DMA,
        ],
    )
    def k(ys_hbm, pa_hbm, pb_hbm, ma_hbm, mb_hbm, out_hbm,
          buf_a, buf_b, idx_a, idx_b, m_a, m_b, sem):
        wid = lax.axis_index("s") * 2 + lax.axis_index("c")
        for sub in range(S // (NW * CROWS)):
            base = wid * (S // NW) + sub * CROWS
            pltpu.sync_copy(pa_hbm.at[pl.ds(base, CROWS)], idx_a)
            pltpu.sync_copy(pb_hbm.at[pl.ds(base, CROWS)], idx_b)
            pltpu.sync_copy(ma_hbm.at[pl.ds(base, CROWS)], m_a)
            pltpu.sync_copy(mb_hbm.at[pl.ds(base, CROWS)], m_b)
            pltpu.async_copy(ys_hbm.at[idx_a], buf_a, sem).wait()
            pltpu.async_copy(ys_hbm.at[idx_b], buf_b, sem).wait()

            for j in range(CROWS):
                ma = m_a[j]
                mb = m_b[j]

                def col_fn(kk, carry2, j=j, ma=ma, mb=mb):
                    a = buf_a[j, pl.ds(kk * 16, 16)]
                    b = buf_b[j, pl.ds(kk * 16, 16)]
                    buf_a[j, pl.ds(kk * 16, 16)] = a * ma + b * mb
                    return carry2

                lax.fori_loop(0, D // 16, col_fn, 0, unroll=8)
            pltpu.sync_copy(buf_a, out_hbm.at[pl.ds(base, CROWS)])

    return k(ys, pos_a, pos_b, m_a_in, m_b_in)


def _gm_body(be_ref, xs_ref, wg_ref, wu_ref, wd_ref, out_ref):
    x = xs_ref[...]
    a = jnp.dot(x, wg_ref[0], preferred_element_type=jnp.float32)
    g = (a * jax.nn.sigmoid(a)) * jnp.dot(x, wu_ref[0], preferred_element_type=jnp.float32)
    out_ref[...] = jnp.dot(g, wd_ref[0], preferred_element_type=jnp.float32)


def _grouped_ffn(be_arr, xs, w_gate, w_up, w_down):
    grid_spec = pltpu.PrefetchScalarGridSpec(
        num_scalar_prefetch=1,
        grid=(NB,),
        in_specs=[
            pl.BlockSpec((MBLK, D), lambda i, be: (i, 0)),
            pl.BlockSpec((1, D, FF), lambda i, be: (be[i], 0, 0)),
            pl.BlockSpec((1, D, FF), lambda i, be: (be[i], 0, 0)),
            pl.BlockSpec((1, FF, D), lambda i, be: (be[i], 0, 0)),
        ],
        out_specs=pl.BlockSpec((MBLK, D), lambda i, be: (i, 0)),
    )
    return pl.pallas_call(
        _gm_body,
        grid_spec=grid_spec,
        out_shape=jax.ShapeDtypeStruct((PADT, D), jnp.float32),
    )(be_arr, xs, w_gate, w_up, w_down)


def _k4_body(sc_ref, comb_ref):
    scores = sc_ref[...]
    iota = jax.lax.broadcasted_iota(jnp.int32, (S, E), 1)
    mlt = jnp.max(scores, axis=-1, keepdims=True)
    idx1 = jnp.min(jnp.where(scores == mlt, iota, E), axis=-1, keepdims=True)
    oh1 = iota == idx1
    factor = jnp.maximum(jnp.abs(scores), mlt)
    mask = ((mlt - scores) / factor) > (2.0 * JITTER)
    mg = jnp.where(mask, NEG, scores)
    m = jnp.max(mg, axis=-1, keepdims=True)
    p = jnp.exp(mg - m)
    sm1 = p / jnp.sum(p, axis=-1, keepdims=True)
    mult1 = jnp.sum(jnp.where(oh1, sm1, 0.0), axis=-1, keepdims=True)

    msc = jnp.where(oh1, NEG, scores)
    mlt2 = jnp.max(msc, axis=-1, keepdims=True)
    idx2 = jnp.min(jnp.where(msc == mlt2, iota, E), axis=-1, keepdims=True)
    oh2 = iota == idx2
    factor2 = jnp.maximum(jnp.abs(scores), mlt2)
    mask2 = ((mlt2 - scores) / factor2) > (2.0 * JITTER)
    mg2 = jnp.where(mask2, NEG, msc)
    m2 = jnp.max(mg2, axis=-1, keepdims=True)
    p2 = jnp.exp(mg2 - m2)
    sm2 = p2 / jnp.sum(p2, axis=-1, keepdims=True)
    mult2 = jnp.sum(jnp.where(oh2, sm2, 0.0), axis=-1, keepdims=True)

    comb_ref[...] = jnp.where(oh1, mult1, 0.0) + jnp.where(oh2, mult2, 0.0)


def _router(logits):
    return pl.pallas_call(
        _k4_body,
        grid=(1,),
        in_specs=[pl.BlockSpec((S, E), lambda i: (0, 0))],
        out_specs=pl.BlockSpec((S, E), lambda i: (0, 0)),
        out_shape=jax.ShapeDtypeStruct((S, E), jnp.float32),
    )(logits)


def _k5_body(x_ref, wg_ref, wu_ref, wd_ref, c_ref, out_ref):
    e = pl.program_id(0)
    f = pl.program_id(1)

    @pl.when(jnp.logical_and(e == 0, f == 0))
    def _():
        out_ref[...] = jnp.zeros_like(out_ref)

    x = x_ref[...]
    a = jnp.dot(x, wg_ref[0], preferred_element_type=jnp.float32)
    g = (a * jax.nn.sigmoid(a)) * jnp.dot(x, wu_ref[0], preferred_element_type=jnp.float32)
    y = jnp.dot(g, wd_ref[0], preferred_element_type=jnp.float32)
    out_ref[...] += c_ref[0] * y


def _moe(xm, w_gate, w_up, w_down, combine_t):
    return pl.pallas_call(
        _k5_body,
        grid=(E, NF),
        in_specs=[
            pl.BlockSpec((S, D), lambda e, f: (0, 0)),
            pl.BlockSpec((1, D, FBLK), lambda e, f: (e, 0, f)),
            pl.BlockSpec((1, D, FBLK), lambda e, f: (e, 0, f)),
            pl.BlockSpec((1, FBLK, D), lambda e, f: (e, f, 0)),
            pl.BlockSpec((1, S, 1), lambda e, f: (e, 0, 0)),
        ],
        out_specs=pl.BlockSpec((S, D), lambda e, f: (0, 0)),
        out_shape=jax.ShapeDtypeStruct((S, D), jnp.float32),
    )(xm, w_gate, w_up, w_down, combine_t)


def kernel(hidden_states, cos, sin, ln1_w, ln2_w, wqkv, wo, gate_w, w_gate, w_up, w_down):
    x = hidden_states.reshape(S, D)
    qkv = _qkv(x, ln1_w, wqkv)
    q3 = qkv[:, : H * HD].reshape(S, H, HD).transpose(1, 0, 2)
    k3 = qkv[:, H * HD: (H + KVH) * HD].reshape(S, KVH, HD).transpose(1, 0, 2)
    v3 = qkv[:, (H + KVH) * HD:].reshape(S, KVH, HD).transpose(1, 0, 2)
    o3 = _attention(q3, k3, v3, cos, sin)
    o2d = o3.transpose(1, 0, 2).reshape(S, H * HD)
    residual2, xm, logits = _oproj_router(o2d, wo, x, ln2_w, gate_w)
    pos3, mult3, be2 = _route_sort(logits)
    posf = pos3.reshape(NA)
    multf = mult3.reshape(NA)
    be_arr = be2.reshape(CHUNK)[:NB]
    xs = _sc_dispatch(xm, posf)
    ys = _grouped_ffn(be_arr, xs, w_gate, w_up, w_down)
    m_a2 = jnp.broadcast_to(multf[:S, None], (S, 16))
    m_b2 = jnp.broadcast_to(multf[S:, None], (S, 16))
    out = _sc_combine(ys, posf[:S], posf[S:], m_a2, m_b2)
    return out.reshape(B, S, D), residual2.reshape(B, S, D)


# speedup vs baseline: 1.4742x; 1.4742x over previous
import functools

import jax
import jax.numpy as jnp
from jax import lax
from jax.experimental import pallas as pl
from jax.experimental.pallas import tpu as pltpu
from jax.experimental.pallas import tpu_sc as plsc

B, S, D = 1, 2048, 1024
H, KVH, HD = 16, 8, 64
E, FF = 8, 2048
EPS = 1e-05
JITTER = 0.01
NEG = -1e30

TBLK = 256
NT = S // TBLK
FBLK = 512
NF = FF // FBLK


def _bdot(a, b):
    return jnp.dot(a.astype(jnp.bfloat16), b.astype(jnp.bfloat16),
                   preferred_element_type=jnp.float32)


def _rmsnorm(x, w):
    return x * jax.lax.rsqrt(jnp.mean(x * x, axis=-1, keepdims=True) + EPS) * w


def _rope(x, cos, sin):
    rot = jnp.concatenate([-x[:, HD // 2:], x[:, :HD // 2]], axis=1)
    return x * cos + rot * sin


def _k1_body(x_ref, w_ref, wqkv_ref, out_ref):
    h = _rmsnorm(x_ref[...], w_ref[...])
    out_ref[...] = _bdot(h, wqkv_ref[...])


def _qkv(x, ln1_w, wqkv):
    return pl.pallas_call(
        _k1_body,
        grid=(NT,),
        in_specs=[
            pl.BlockSpec((TBLK, D), lambda i: (i, 0)),
            pl.BlockSpec((1, D), lambda i: (0, 0)),
            pl.BlockSpec((D, (H + 2 * KVH) * HD), lambda i: (0, 0)),
        ],
        out_specs=pl.BlockSpec((TBLK, (H + 2 * KVH) * HD), lambda i: (i, 0)),
        out_shape=jax.ShapeDtypeStruct((S, (H + 2 * KVH) * HD), jnp.float32),
    )(x, ln1_w.reshape(1, D), wqkv)


def _k2_body(q_ref, k_ref, v_ref, cosq_ref, sinq_ref, cos_ref, sin_ref, o_ref):
    i = pl.program_id(1)
    q = _rope(q_ref[0], cosq_ref[...], sinq_ref[...])
    k = _rope(k_ref[0], cos_ref[...], sin_ref[...])
    logits = jax.lax.dot_general(
        q.astype(jnp.bfloat16), k.astype(jnp.bfloat16),
        (((1,), (1,)), ((), ())), preferred_element_type=jnp.float32)
    logits = logits * (1.0 / (HD ** 0.5))
    qi = i * TBLK + jax.lax.broadcasted_iota(jnp.int32, (TBLK, S), 0)
    kj = jax.lax.broadcasted_iota(jnp.int32, (TBLK, S), 1)
    logits = jnp.where(kj <= qi, logits, NEG)
    m = jnp.max(logits, axis=-1, keepdims=True)
    p = jnp.exp(logits - m)
    p = p / jnp.sum(p, axis=-1, keepdims=True)
    o_ref[0] = _bdot(p, v_ref[0])


def _attention(q3, k3, v3, cos, sin):
    return pl.pallas_call(
        _k2_body,
        grid=(H, NT),
        in_specs=[
            pl.BlockSpec((1, TBLK, HD), lambda h, i: (h, i, 0)),
            pl.BlockSpec((1, S, HD), lambda h, i: (h // (H // KVH), 0, 0)),
            pl.BlockSpec((1, S, HD), lambda h, i: (h // (H // KVH), 0, 0)),
            pl.BlockSpec((TBLK, HD), lambda h, i: (i, 0)),
            pl.BlockSpec((TBLK, HD), lambda h, i: (i, 0)),
            pl.BlockSpec((S, HD), lambda h, i: (0, 0)),
            pl.BlockSpec((S, HD), lambda h, i: (0, 0)),
        ],
        out_specs=pl.BlockSpec((1, TBLK, HD), lambda h, i: (h, i, 0)),
        out_shape=jax.ShapeDtypeStruct((H, S, HD), jnp.float32),
    )(q3, k3, v3, cos, sin, cos, sin)


def _k3_body(o_ref, wo_ref, res_ref, w2_ref, gw_ref, r2_ref, xm_ref, lg_ref):
    r2 = _bdot(o_ref[...], wo_ref[...]) + res_ref[...]
    r2_ref[...] = r2
    h2 = _rmsnorm(r2, w2_ref[...])
    xm_ref[...] = h2
    lg_ref[...] = _bdot(h2, gw_ref[...])


def _oproj_router(o2d, wo, resid, ln2_w, gate_w):
    return pl.pallas_call(
        _k3_body,
        grid=(NT,),
        in_specs=[
            pl.BlockSpec((TBLK, H * HD), lambda i: (i, 0)),
            pl.BlockSpec((H * HD, D), lambda i: (0, 0)),
            pl.BlockSpec((TBLK, D), lambda i: (i, 0)),
            pl.BlockSpec((1, D), lambda i: (0, 0)),
            pl.BlockSpec((D, E), lambda i: (0, 0)),
        ],
        out_specs=[
            pl.BlockSpec((TBLK, D), lambda i: (i, 0)),
            pl.BlockSpec((TBLK, D), lambda i: (i, 0)),
            pl.BlockSpec((TBLK, E), lambda i: (i, 0)),
        ],
        out_shape=[
            jax.ShapeDtypeStruct((S, D), jnp.float32),
            jax.ShapeDtypeStruct((S, D), jnp.float32),
            jax.ShapeDtypeStruct((S, E), jnp.float32),
        ],
    )(o2d, wo, resid, ln2_w.reshape(1, D), gate_w)


NA = 2 * S
MBLK = 128
NB = 40
PADT = NB * MBLK
CHUNK = 128
NCH = NA // CHUNK


def _sparsemixer_t(sc):
    iota_e = lax.broadcasted_iota(jnp.int32, (E, CHUNK), 0)
    mlt = jnp.max(sc, axis=0, keepdims=True)
    idx1 = jnp.min(jnp.where(sc == mlt, iota_e, E), axis=0, keepdims=True)
    oh1 = iota_e == idx1
    factor = jnp.maximum(jnp.abs(sc), mlt)
    mask = ((mlt - sc) / factor) > (2.0 * JITTER)
    mg = jnp.where(mask, NEG, sc)
    p1 = jnp.exp(mg - jnp.max(mg, axis=0, keepdims=True))
    sm1 = p1 / jnp.sum(p1, axis=0, keepdims=True)
    mult1 = jnp.sum(jnp.where(oh1, sm1, 0.0), axis=0, keepdims=True)

    msc = jnp.where(oh1, NEG, sc)
    mlt2 = jnp.max(msc, axis=0, keepdims=True)
    idx2 = jnp.min(jnp.where(msc == mlt2, iota_e, E), axis=0, keepdims=True)
    oh2 = iota_e == idx2
    factor2 = jnp.maximum(jnp.abs(sc), mlt2)
    mask2 = ((mlt2 - sc) / factor2) > (2.0 * JITTER)
    mg2 = jnp.where(mask2, NEG, msc)
    p2 = jnp.exp(mg2 - jnp.max(mg2, axis=0, keepdims=True))
    sm2 = p2 / jnp.sum(p2, axis=0, keepdims=True)
    mult2 = jnp.sum(jnp.where(oh2, sm2, 0.0), axis=0, keepdims=True)
    return oh1.astype(jnp.float32), oh2.astype(jnp.float32), mult1, mult2


def _k4b_body(sct_ref, pos_ref, mult_ref, be_ref, cnt_ref, pst_ref):
    p = pl.program_id(0)
    c = pl.program_id(1)
    oh1, oh2, mult1, mult2 = _sparsemixer_t(sct_ref[...])
    is1 = c < (NCH // 2)
    mc = jnp.where(is1, oh1, oh2)
    multc = jnp.where(is1, mult1, mult2)
    colsum = jnp.sum(mc, axis=1, keepdims=True)

    @pl.when(c == 0)
    def _():
        cnt_ref[...] = jnp.zeros_like(cnt_ref)

    @pl.when(p == 0)
    def _():
        cnt_ref[...] += jnp.broadcast_to(colsum, (E, CHUNK))

        @pl.when(c == NCH - 1)
        def _():
            blk = jnp.floor((cnt_ref[...] + (MBLK - 1.0)) * (1.0 / MBLK))
            acc = jnp.zeros((1, CHUNK), jnp.float32)
            for e in range(E):
                pst_ref[e:e + 1, :] = acc
                acc = acc + blk[e:e + 1, :]
            iota_b = lax.broadcasted_iota(jnp.int32, (1, CHUNK), 1)
            bev = jnp.zeros((1, CHUNK), jnp.int32)
            for e in range(E):
                incl = pst_ref[e:e + 1, 0:1] + blk[e:e + 1, 0:1]
                incl_b = jnp.broadcast_to(incl, (1, CHUNK)).astype(jnp.int32)
                bev = bev + jnp.where(iota_b >= incl_b, 1, 0)
            be_ref[...] = jnp.minimum(bev, E - 1)

    @pl.when(p == 1)
    def _():
        iota_r = lax.broadcasted_iota(jnp.int32, (CHUNK, CHUNK), 0)
        iota_c = lax.broadcasted_iota(jnp.int32, (CHUNK, CHUNK), 1)
        tri = jnp.where(iota_r < iota_c, 1.0, 0.0)
        prefix = jnp.dot(mc, tri, preferred_element_type=jnp.float32)
        rank = prefix + jnp.broadcast_to(cnt_ref[:, 0:1], (E, CHUNK))
        base = pst_ref[...] * float(MBLK)
        posv = jnp.sum(mc * (rank + base), axis=0, keepdims=True)
        pos_ref[0] = posv.astype(jnp.int32)
        mult_ref[0] = multc
        cnt_ref[...] += jnp.broadcast_to(colsum, (E, CHUNK))


def _route_sort(logits):
    return pl.pallas_call(
        _k4b_body,
        grid=(2, NCH),
        in_specs=[pl.BlockSpec((E, CHUNK), lambda p, c: (0, lax.rem(c, NCH // 2)))],
        out_specs=[
            pl.BlockSpec((1, 1, CHUNK), lambda p, c: (c, 0, 0)),
            pl.BlockSpec((1, 1, CHUNK), lambda p, c: (c, 0, 0)),
            pl.BlockSpec((1, CHUNK), lambda p, c: (0, 0)),
        ],
        out_shape=[
            jax.ShapeDtypeStruct((NCH, 1, CHUNK), jnp.int32),
            jax.ShapeDtypeStruct((NCH, 1, CHUNK), jnp.float32),
            jax.ShapeDtypeStruct((1, CHUNK), jnp.int32),
        ],
        scratch_shapes=[
            pltpu.VMEM((E, CHUNK), jnp.float32),
            pltpu.VMEM((E, CHUNK), jnp.float32),
        ],
    )(logits.T)


NW = 32
DROWS = 64
CROWS = 32


def _sc_dispatch(xm, posf):
    mesh = plsc.VectorSubcoreMesh(core_axis_name="c", subcore_axis_name="s")

    @functools.partial(
        pl.kernel, mesh=mesh,
        out_type=jax.ShapeDtypeStruct((PADT, D), jnp.float32),
        scratch_types=[
            pltpu.VMEM((DROWS, D), jnp.float32),
            pltpu.VMEM((DROWS,), jnp.int32),
            pltpu.SemaphoreType.DMA,
        ],
    )
    def k(xm_hbm, posf_hbm, xs_hbm, rows_v, idx_v, sem):
        wid = lax.axis_index("s") * 2 + lax.axis_index("c")
        for j in range(NA // (NW * DROWS)):
            q = wid * 2 + j
            base_t = lax.rem(q * DROWS, S)
            pltpu.sync_copy(xm_hbm.at[pl.ds(base_t, DROWS)], rows_v)
            pltpu.sync_copy(posf_hbm.at[pl.ds(q * DROWS, DROWS)], idx_v)
            pltpu.async_copy(rows_v, xs_hbm.at[idx_v], sem).wait()

    return k(xm, posf)


def _sc_combine(ys, pos_a, pos_b, m_a_in, m_b_in):
    mesh = plsc.VectorSubcoreMesh(core_axis_name="c", subcore_axis_name="s")

    @functools.partial(
        pl.kernel, mesh=mesh,
        out_type=jax.ShapeDtypeStruct((S, D), jnp.float32),
        scratch_types=[
            pltpu.VMEM((CROWS, D), jnp.float32),
            pltpu.VMEM((CROWS, D), jnp.float32),
            pltpu.VMEM((CROWS,), jnp.int32),
            pltpu.VMEM((CROWS,), jnp.int32),
            pltpu.VMEM((CROWS, 16), jnp.float32),
            pltpu.VMEM((CROWS, 16), jnp.float32),
            pltpu.SemaphoreType.DMA,
        ],
    )
    def k(ys_hbm, pa_hbm, pb_hbm, ma_hbm, mb_hbm, out_hbm,
          buf_a, buf_b, idx_a, idx_b, m_a, m_b, sem):
        wid = lax.axis_index("s") * 2 + lax.axis_index("c")
        for sub in range(S // (NW * CROWS)):
            base = wid * (S // NW) + sub * CROWS
            pltpu.sync_copy(pa_hbm.at[pl.ds(base, CROWS)], idx_a)
            pltpu.sync_copy(pb_hbm.at[pl.ds(base, CROWS)], idx_b)
            pltpu.sync_copy(ma_hbm.at[pl.ds(base, CROWS)], m_a)
            pltpu.sync_copy(mb_hbm.at[pl.ds(base, CROWS)], m_b)
            pltpu.async_copy(ys_hbm.at[idx_a], buf_a, sem).wait()
            pltpu.async_copy(ys_hbm.at[idx_b], buf_b, sem).wait()

            for j in range(CROWS):
                ma = m_a[j]
                mb = m_b[j]

                def col_fn(kk, carry2, j=j, ma=ma, mb=mb):
                    a = buf_a[j, pl.ds(kk * 16, 16)]
                    b = buf_b[j, pl.ds(kk * 16, 16)]
                    buf_a[j, pl.ds(kk * 16, 16)] = a * ma + b * mb
                    return carry2

                lax.fori_loop(0, D // 16, col_fn, 0, unroll=8)
            pltpu.sync_copy(buf_a, out_hbm.at[pl.ds(base, CROWS)])

    return k(ys, pos_a, pos_b, m_a_in, m_b_in)


def _gm_body(be_ref, xs_ref, wg_ref, wu_ref, wd_ref, out_ref):
    x = xs_ref[...]
    a = _bdot(x, wg_ref[0])
    g = (a * jax.nn.sigmoid(a)) * _bdot(x, wu_ref[0])
    out_ref[...] = _bdot(g, wd_ref[0])


def _grouped_ffn(be_arr, xs, w_gate, w_up, w_down):
    grid_spec = pltpu.PrefetchScalarGridSpec(
        num_scalar_prefetch=1,
        grid=(NB,),
        in_specs=[
            pl.BlockSpec((MBLK, D), lambda i, be: (i, 0)),
            pl.BlockSpec((1, D, FF), lambda i, be: (be[i], 0, 0)),
            pl.BlockSpec((1, D, FF), lambda i, be: (be[i], 0, 0)),
            pl.BlockSpec((1, FF, D), lambda i, be: (be[i], 0, 0)),
        ],
        out_specs=pl.BlockSpec((MBLK, D), lambda i, be: (i, 0)),
    )
    return pl.pallas_call(
        _gm_body,
        grid_spec=grid_spec,
        out_shape=jax.ShapeDtypeStruct((PADT, D), jnp.float32),
    )(be_arr, xs, w_gate, w_up, w_down)


def _k4_body(sc_ref, comb_ref):
    scores = sc_ref[...]
    iota = jax.lax.broadcasted_iota(jnp.int32, (S, E), 1)
    mlt = jnp.max(scores, axis=-1, keepdims=True)
    idx1 = jnp.min(jnp.where(scores == mlt, iota, E), axis=-1, keepdims=True)
    oh1 = iota == idx1
    factor = jnp.maximum(jnp.abs(scores), mlt)
    mask = ((mlt - scores) / factor) > (2.0 * JITTER)
    mg = jnp.where(mask, NEG, scores)
    m = jnp.max(mg, axis=-1, keepdims=True)
    p = jnp.exp(mg - m)
    sm1 = p / jnp.sum(p, axis=-1, keepdims=True)
    mult1 = jnp.sum(jnp.where(oh1, sm1, 0.0), axis=-1, keepdims=True)

    msc = jnp.where(oh1, NEG, scores)
    mlt2 = jnp.max(msc, axis=-1, keepdims=True)
    idx2 = jnp.min(jnp.where(msc == mlt2, iota, E), axis=-1, keepdims=True)
    oh2 = iota == idx2
    factor2 = jnp.maximum(jnp.abs(scores), mlt2)
    mask2 = ((mlt2 - scores) / factor2) > (2.0 * JITTER)
    mg2 = jnp.where(mask2, NEG, msc)
    m2 = jnp.max(mg2, axis=-1, keepdims=True)
    p2 = jnp.exp(mg2 - m2)
    sm2 = p2 / jnp.sum(p2, axis=-1, keepdims=True)
    mult2 = jnp.sum(jnp.where(oh2, sm2, 0.0), axis=-1, keepdims=True)

    comb_ref[...] = jnp.where(oh1, mult1, 0.0) + jnp.where(oh2, mult2, 0.0)


def _router(logits):
    return pl.pallas_call(
        _k4_body,
        grid=(1,),
        in_specs=[pl.BlockSpec((S, E), lambda i: (0, 0))],
        out_specs=pl.BlockSpec((S, E), lambda i: (0, 0)),
        out_shape=jax.ShapeDtypeStruct((S, E), jnp.float32),
    )(logits)


def _k5_body(x_ref, wg_ref, wu_ref, wd_ref, c_ref, out_ref):
    e = pl.program_id(0)
    f = pl.program_id(1)

    @pl.when(jnp.logical_and(e == 0, f == 0))
    def _():
        out_ref[...] = jnp.zeros_like(out_ref)

    x = x_ref[...]
    a = jnp.dot(x, wg_ref[0], preferred_element_type=jnp.float32)
    g = (a * jax.nn.sigmoid(a)) * jnp.dot(x, wu_ref[0], preferred_element_type=jnp.float32)
    y = jnp.dot(g, wd_ref[0], preferred_element_type=jnp.float32)
    out_ref[...] += c_ref[0] * y


def _moe(xm, w_gate, w_up, w_down, combine_t):
    return pl.pallas_call(
        _k5_body,
        grid=(E, NF),
        in_specs=[
            pl.BlockSpec((S, D), lambda e, f: (0, 0)),
            pl.BlockSpec((1, D, FBLK), lambda e, f: (e, 0, f)),
            pl.BlockSpec((1, D, FBLK), lambda e, f: (e, 0, f)),
            pl.BlockSpec((1, FBLK, D), lambda e, f: (e, f, 0)),
            pl.BlockSpec((1, S, 1), lambda e, f: (e, 0, 0)),
        ],
        out_specs=pl.BlockSpec((S, D), lambda e, f: (0, 0)),
        out_shape=jax.ShapeDtypeStruct((S, D), jnp.float32),
    )(xm, w_gate, w_up, w_down, combine_t)


def kernel(hidden_states, cos, sin, ln1_w, ln2_w, wqkv, wo, gate_w, w_gate, w_up, w_down):
    x = hidden_states.reshape(S, D)
    qkv = _qkv(x, ln1_w, wqkv)
    q3 = qkv[:, : H * HD].reshape(S, H, HD).transpose(1, 0, 2)
    k3 = qkv[:, H * HD: (H + KVH) * HD].reshape(S, KVH, HD).transpose(1, 0, 2)
    v3 = qkv[:, (H + KVH) * HD:].reshape(S, KVH, HD).transpose(1, 0, 2)
    o3 = _attention(q3, k3, v3, cos, sin)
    o2d = o3.transpose(1, 0, 2).reshape(S, H * HD)
    residual2, xm, logits = _oproj_router(o2d, wo, x, ln2_w, gate_w)
    pos3, mult3, be2 = _route_sort(logits)
    posf = pos3.reshape(NA)
    multf = mult3.reshape(NA)
    be_arr = be2.reshape(CHUNK)[:NB]
    xs = _sc_dispatch(xm, posf)
    ys = _grouped_ffn(be_arr, xs, w_gate, w_up, w_down)
    m_a2 = jnp.broadcast_to(multf[:S, None], (S, 16))
    m_b2 = jnp.broadcast_to(multf[S:, None], (S, 16))
    out = _sc_combine(ys, posf[:S], posf[S:], m_a2, m_b2)
    return out.reshape(B, S, D), residual2.reshape(B, S, D)
